# baseline (speedup 1.0000x reference)
import functools

import numpy as np
import jax
import jax.numpy as jnp
from jax.experimental import pallas as pl
from jax.experimental.pallas import tpu as pltpu


def _round_up(v, m):
    return (v + m - 1) // m * m


def _pick(dim, cands):
    for c in cands:
        if dim % c == 0:
            return c
    raise ValueError(f"no block size for {dim} in {cands}")


_LANE = (512, 256, 128)
_ROW = (512, 400, 256, 200, 128, 80, 40, 8)


def _mm_kernel(a_ref, b_ref, o_ref, acc_ref, *, nk, zero_diag, bm, bn):
    k = pl.program_id(2)

    @pl.when(k == 0)
    def _():
        acc_ref[...] = jnp.zeros_like(acc_ref)

    acc_ref[...] += jnp.dot(a_ref[...], b_ref[...],
                            preferred_element_type=jnp.float32)

    @pl.when(k == nk - 1)
    def _():
        out = acc_ref[...]
        if zero_diag:
            i, j = pl.program_id(0), pl.program_id(1)
            rows = jax.lax.broadcasted_iota(jnp.int32, (bm, bn), 0) + i * bm
            cols = jax.lax.broadcasted_iota(jnp.int32, (bm, bn), 1) + j * bn
            out = jnp.where(rows == cols, 0.0, out)
        o_ref[...] = out


def _mm(a, b, zero_diag=False):
    m, kdim = a.shape
    _, n = b.shape
    bm = _pick(m, (512, 256, 128, 8))
    bn = _pick(n, _LANE)
    bk = _pick(kdim, _LANE)
    grid = (m // bm, n // bn, kdim // bk)
    return pl.pallas_call(
        functools.partial(_mm_kernel, nk=grid[2], zero_diag=zero_diag,
                          bm=bm, bn=bn),
        grid=grid,
        in_specs=[
            pl.BlockSpec((bm, bk), lambda i, j, k: (i, k)),
            pl.BlockSpec((bk, bn), lambda i, j, k: (k, j)),
        ],
        out_specs=pl.BlockSpec((bm, bn), lambda i, j, k: (i, j)),
        out_shape=jax.ShapeDtypeStruct((m, n), jnp.float32),
        scratch_shapes=[pltpu.VMEM((bm, bn), jnp.float32)],
        compiler_params=pltpu.CompilerParams(
            dimension_semantics=("parallel", "parallel", "arbitrary")),
    )(a, b)


def _xw_kernel(x_ref, w_ref, dis_ref, o_ref):
    o_ref[...] = dis_ref[...] * jnp.dot(x_ref[...], w_ref[...],
                                        preferred_element_type=jnp.float32)


def _xw_scale(x, w, dis):
    m, d = x.shape
    h = w.shape[1]
    bm = _pick(m, _ROW)
    return pl.pallas_call(
        _xw_kernel,
        grid=(m // bm,),
        in_specs=[
            pl.BlockSpec((bm, d), lambda i: (i, 0)),
            pl.BlockSpec((d, h), lambda i: (0, 0)),
            pl.BlockSpec((bm, 1), lambda i: (i, 0)),
        ],
        out_specs=pl.BlockSpec((bm, h), lambda i: (i, 0)),
        out_shape=jax.ShapeDtypeStruct((m, h), jnp.float32),
        compiler_params=pltpu.CompilerParams(
            dimension_semantics=("parallel",)),
    )(x, w, dis[:, None])


def _agg_kernel(a_ref, tk_ref, ti_ref, dis_ref, b_ref, o_ref, acc_ref,
                *, nk, relu):
    k = pl.program_id(1)

    @pl.when(k == 0)
    def _():
        acc_ref[...] = jnp.zeros_like(acc_ref)

    acc_ref[...] += jax.lax.dot_general(
        a_ref[...], tk_ref[...], (((0,), (0,)), ((), ())),
        preferred_element_type=jnp.float32)

    @pl.when(k == nk - 1)
    def _():
        out = dis_ref[...] * (acc_ref[...] + 2.0 * ti_ref[...]) + b_ref[...]
        o_ref[...] = jnp.maximum(out, 0.0) if relu else out


def _gcn_agg(A, t, dis, b, relu):
    m = A.shape[0]
    h = t.shape[1]
    bm = _pick(m, _LANE)
    bk = _pick(m, _LANE)
    grid = (m // bm, m // bk)
    return pl.pallas_call(
        functools.partial(_agg_kernel, nk=grid[1], relu=relu),
        grid=grid,
        in_specs=[
            pl.BlockSpec((bk, bm), lambda i, k: (k, i)),
            pl.BlockSpec((bk, h), lambda i, k: (k, 0)),
            pl.BlockSpec((bm, h), lambda i, k: (i, 0)),
            pl.BlockSpec((bm, 1), lambda i, k: (i, 0)),
            pl.BlockSpec((1, h), lambda i, k: (0, 0)),
        ],
        out_specs=pl.BlockSpec((bm, h), lambda i, k: (i, 0)),
        out_shape=jax.ShapeDtypeStruct((m, h), jnp.float32),
        scratch_shapes=[pltpu.VMEM((bm, h), jnp.float32)],
        compiler_params=pltpu.CompilerParams(
            dimension_semantics=("parallel", "arbitrary")),
    )(A, t, t, dis[:, None], b[None, :])


def kernel(x, edge_index, down_W, down_b, pool_p, up_W, up_b):
    n0, _ = x.shape
    e = edge_index.shape[1]
    src, dst = edge_index[0], edge_index[1]

    ones_e = jnp.ones((e,), jnp.float32)
    cnt_dst = jax.ops.segment_sum(ones_e, dst, num_segments=n0)
    self_cnt = jax.ops.segment_sum((src == dst).astype(jnp.float32), dst,
                                   num_segments=n0)
    selfw = jnp.where(self_cnt == 0.0, 2.0, 0.0)
    deg0 = cnt_dst + selfw
    dis0 = jnp.where(deg0 > 0.0, jax.lax.rsqrt(deg0), 0.0)

    def gcn0(xin, W, b, relu):
        t = _xw_scale(xin, W, dis0)
        seg = jax.ops.segment_sum(jnp.take(t, src, axis=0), dst,
                                  num_segments=n0)
        out = dis0[:, None] * (seg + selfw[:, None] * t) + b[None, :]
        return jnp.maximum(out, 0.0) if relu else out

    def pool_sel(h, nreal, p, know):
        score = jnp.tanh(jnp.dot(h[:nreal], p) / jnp.linalg.norm(p))
        vals, perm = jax.lax.top_k(score, know)
        return vals, perm

    k1 = int(np.ceil(0.5 * n0))
    k2 = int(np.ceil(0.5 * k1))
    k3 = int(np.ceil(0.5 * k2))
    p1, p2, p3 = _round_up(k1, 128), _round_up(k2, 128), _round_up(k3, 128)
    pk0 = _round_up(n0, 128)

    h0 = gcn0(x, down_W[0], down_b[0], relu=True)

    vals1, perm1 = pool_sel(h0, n0, pool_p[0], k1)
    inv1 = jnp.full((n0,), -1, jnp.int32).at[perm1].set(
        jnp.arange(k1, dtype=jnp.int32))
    nonself = src != dst
    okB = nonself & (inv1[dst] >= 0)
    colB = jnp.where(okB, inv1[dst], p1)
    B1 = jnp.zeros((pk0, p1), jnp.float32).at[src, colB].add(1.0, mode="drop")
    B1 = B1.at[perm1, jnp.arange(k1)].set(1.0)
    okS = nonself & (inv1[src] >= 0)
    rowS = jnp.where(okS, inv1[src], p1)
    S1 = jnp.zeros((p1, pk0), jnp.float32).at[rowS, dst].add(1.0, mode="drop")
    S1 = S1.at[jnp.arange(k1), perm1].set(1.0)
    A1 = _mm(S1, B1, zero_diag=True)

    deg1 = _mm(jnp.ones((8, p1), jnp.float32), A1)[0] + 2.0
    dis1 = jax.lax.rsqrt(deg1)
    x1 = jnp.zeros((p1, h0.shape[1]), jnp.float32).at[:k1].set(
        jnp.take(h0, perm1, axis=0) * vals1[:, None])
    h1 = _gcn_agg(A1, _xw_scale(x1, down_W[1], dis1), dis1, down_b[1],
                  relu=True)

    def next_A(A, perm, know, pnow):
        idx = jnp.arange(know)
        Sr = jnp.take(A, perm, axis=0).at[idx, perm].set(1.0)
        S = jnp.zeros((pnow, A.shape[0]), jnp.float32).at[:know].set(Sr)
        Bc = jnp.take(A, perm, axis=1).at[perm, idx].set(1.0)
        B = jnp.zeros((A.shape[0], pnow), jnp.float32).at[:, :know].set(Bc)
        return _mm(S, B, zero_diag=True)

    vals2, perm2 = pool_sel(h1, k1, pool_p[1], k2)
    A2 = next_A(A1, perm2, k2, p2)
    deg2 = _mm(jnp.ones((8, p2), jnp.float32), A2)[0] + 2.0
    dis2 = jax.lax.rsqrt(deg2)
    x2 = jnp.zeros((p2, h1.shape[1]), jnp.float32).at[:k2].set(
        jnp.take(h1, perm2, axis=0) * vals2[:, None])
    h2 = _gcn_agg(A2, _xw_scale(x2, down_W[2], dis2), dis2, down_b[2],
                  relu=True)

    vals3, perm3 = pool_sel(h2, k2, pool_p[2], k3)
    A3 = next_A(A2, perm3, k3, p3)
    deg3 = _mm(jnp.ones((8, p3), jnp.float32), A3)[0] + 2.0
    dis3 = jax.lax.rsqrt(deg3)
    x3 = jnp.zeros((p3, h2.shape[1]), jnp.float32).at[:k3].set(
        jnp.take(h2, perm3, axis=0) * vals3[:, None])
    h3 = _gcn_agg(A3, _xw_scale(x3, down_W[3], dis3), dis3, down_b[3],
                  relu=True)

    xu = h2 + jnp.zeros_like(h2).at[perm3].set(h3[:k3])
    hu2 = _gcn_agg(A2, _xw_scale(xu, up_W[0], dis2), dis2, up_b[0],
                   relu=True)
    xu = h1 + jnp.zeros_like(h1).at[perm2].set(hu2[:k2])
    hu1 = _gcn_agg(A1, _xw_scale(xu, up_W[1], dis1), dis1, up_b[1],
                   relu=True)
    xu = h0 + jnp.zeros_like(h0).at[perm1].set(hu1[:k1])
    return gcn0(xu, up_W[2], up_b[2], relu=False)


# speedup vs baseline: 1.1497x; 1.1497x over previous
import functools

import numpy as np
import jax
import jax.numpy as jnp
from jax.experimental import pallas as pl
from jax.experimental.pallas import tpu as pltpu


def _round_up(v, m):
    return (v + m - 1) // m * m


def _pick(dim, cands):
    for c in cands:
        if dim % c == 0:
            return c
    raise ValueError(f"no block size for {dim} in {cands}")


_LANE = (2048, 1024, 512, 256, 128)
_MMROW = (1024, 512, 256, 128, 8)
_ROW = (512, 400, 256, 200, 128, 80, 40, 8)


def _mm_kernel(a_ref, b_ref, o_ref, acc_ref, *, nk, zero_diag, bm, bn):
    k = pl.program_id(2)

    @pl.when(k == 0)
    def _():
        acc_ref[...] = jnp.zeros_like(acc_ref)

    acc_ref[...] += jnp.dot(a_ref[...], b_ref[...],
                            preferred_element_type=jnp.float32)

    @pl.when(k == nk - 1)
    def _():
        out = acc_ref[...]
        if zero_diag:
            i, j = pl.program_id(0), pl.program_id(1)
            rows = jax.lax.broadcasted_iota(jnp.int32, (bm, bn), 0) + i * bm
            cols = jax.lax.broadcasted_iota(jnp.int32, (bm, bn), 1) + j * bn
            out = jnp.where(rows == cols, 0.0, out)
        o_ref[...] = out


def _mm(a, b, zero_diag=False):
    m, kdim = a.shape
    _, n = b.shape
    bm = _pick(m, _MMROW)
    bn = _pick(n, _LANE)
    bk = _pick(kdim, _LANE)
    grid = (m // bm, n // bn, kdim // bk)
    return pl.pallas_call(
        functools.partial(_mm_kernel, nk=grid[2], zero_diag=zero_diag,
                          bm=bm, bn=bn),
        grid=grid,
        in_specs=[
            pl.BlockSpec((bm, bk), lambda i, j, k: (i, k)),
            pl.BlockSpec((bk, bn), lambda i, j, k: (k, j)),
        ],
        out_specs=pl.BlockSpec((bm, bn), lambda i, j, k: (i, j)),
        out_shape=jax.ShapeDtypeStruct((m, n), jnp.float32),
        scratch_shapes=[pltpu.VMEM((bm, bn), jnp.float32)],
        compiler_params=pltpu.CompilerParams(
            dimension_semantics=("parallel", "parallel", "arbitrary")),
    )(a, b)


def _xw_kernel(x_ref, w_ref, dis_ref, o_ref):
    o_ref[...] = dis_ref[...] * jnp.dot(x_ref[...], w_ref[...],
                                        preferred_element_type=jnp.float32)


def _xw_scale(x, w, dis):
    m, d = x.shape
    h = w.shape[1]
    bm = _pick(m, _ROW)
    return pl.pallas_call(
        _xw_kernel,
        grid=(m // bm,),
        in_specs=[
            pl.BlockSpec((bm, d), lambda i: (i, 0)),
            pl.BlockSpec((d, h), lambda i: (0, 0)),
            pl.BlockSpec((bm, 1), lambda i: (i, 0)),
        ],
        out_specs=pl.BlockSpec((bm, h), lambda i: (i, 0)),
        out_shape=jax.ShapeDtypeStruct((m, h), jnp.float32),
        compiler_params=pltpu.CompilerParams(
            dimension_semantics=("parallel",)),
    )(x, w, dis[:, None])


def _agg_kernel(a_ref, tk_ref, ti_ref, dis_ref, b_ref, o_ref, acc_ref,
                *, nk, relu):
    k = pl.program_id(1)

    @pl.when(k == 0)
    def _():
        acc_ref[...] = jnp.zeros_like(acc_ref)

    acc_ref[...] += jax.lax.dot_general(
        a_ref[...], tk_ref[...], (((0,), (0,)), ((), ())),
        preferred_element_type=jnp.float32)

    @pl.when(k == nk - 1)
    def _():
        out = dis_ref[...] * (acc_ref[...] + 2.0 * ti_ref[...]) + b_ref[...]
        o_ref[...] = jnp.maximum(out, 0.0) if relu else out


def _gcn_agg(A, t, dis, b, relu):
    m = A.shape[0]
    h = t.shape[1]
    bm = _pick(m, _LANE)
    bk = _pick(m, _LANE)
    grid = (m // bm, m // bk)
    return pl.pallas_call(
        functools.partial(_agg_kernel, nk=grid[1], relu=relu),
        grid=grid,
        in_specs=[
            pl.BlockSpec((bk, bm), lambda i, k: (k, i)),
            pl.BlockSpec((bk, h), lambda i, k: (k, 0)),
            pl.BlockSpec((bm, h), lambda i, k: (i, 0)),
            pl.BlockSpec((bm, 1), lambda i, k: (i, 0)),
            pl.BlockSpec((1, h), lambda i, k: (0, 0)),
        ],
        out_specs=pl.BlockSpec((bm, h), lambda i, k: (i, 0)),
        out_shape=jax.ShapeDtypeStruct((m, h), jnp.float32),
        scratch_shapes=[pltpu.VMEM((bm, h), jnp.float32)],
        compiler_params=pltpu.CompilerParams(
            dimension_semantics=("parallel", "arbitrary")),
    )(A, t, t, dis[:, None], b[None, :])


def kernel(x, edge_index, down_W, down_b, pool_p, up_W, up_b):
    n0, _ = x.shape
    e = edge_index.shape[1]
    src, dst = edge_index[0], edge_index[1]

    ones_e = jnp.ones((e,), jnp.float32)
    cnt_dst = jax.ops.segment_sum(ones_e, dst, num_segments=n0)
    self_cnt = jax.ops.segment_sum((src == dst).astype(jnp.float32), dst,
                                   num_segments=n0)
    selfw = jnp.where(self_cnt == 0.0, 2.0, 0.0)
    deg0 = cnt_dst + selfw
    dis0 = jnp.where(deg0 > 0.0, jax.lax.rsqrt(deg0), 0.0)

    def gcn0(xin, W, b, relu):
        t = _xw_scale(xin, W, dis0)
        seg = jax.ops.segment_sum(jnp.take(t, src, axis=0), dst,
                                  num_segments=n0)
        out = dis0[:, None] * (seg + selfw[:, None] * t) + b[None, :]
        return jnp.maximum(out, 0.0) if relu else out

    def pool_sel(h, nreal, p, know):
        score = jnp.tanh(jnp.dot(h[:nreal], p) / jnp.linalg.norm(p))
        vals, perm = jax.lax.top_k(score, know)
        return vals, perm

    k1 = int(np.ceil(0.5 * n0))
    k2 = int(np.ceil(0.5 * k1))
    k3 = int(np.ceil(0.5 * k2))
    p1, p2, p3 = _round_up(k1, 128), _round_up(k2, 128), _round_up(k3, 128)
    pk0 = _round_up(n0, 128)

    h0 = gcn0(x, down_W[0], down_b[0], relu=True)

    vals1, perm1 = pool_sel(h0, n0, pool_p[0], k1)
    inv1 = jnp.full((n0,), -1, jnp.int32).at[perm1].set(
        jnp.arange(k1, dtype=jnp.int32))
    nonself = src != dst
    okB = nonself & (inv1[dst] >= 0)
    colB = jnp.where(okB, inv1[dst], p1)
    B1 = jnp.zeros((pk0, p1), jnp.bfloat16).at[src, colB].add(1.0, mode="drop")
    B1 = B1.at[perm1, jnp.arange(k1)].set(1.0)
    okS = nonself & (inv1[src] >= 0)
    rowS = jnp.where(okS, inv1[src], p1)
    S1 = jnp.zeros((p1, pk0), jnp.bfloat16).at[rowS, dst].add(1.0, mode="drop")
    S1 = S1.at[jnp.arange(k1), perm1].set(1.0)
    A1 = _mm(S1, B1, zero_diag=True)

    deg1 = _mm(jnp.ones((8, p1), jnp.float32), A1)[0] + 2.0
    dis1 = jax.lax.rsqrt(deg1)
    x1 = jnp.zeros((p1, h0.shape[1]), jnp.float32).at[:k1].set(
        jnp.take(h0, perm1, axis=0) * vals1[:, None])
    h1 = _gcn_agg(A1, _xw_scale(x1, down_W[1], dis1), dis1, down_b[1],
                  relu=True)

    def next_A(A, perm, know, pnow):
        idx = jnp.arange(know)
        Sr = jnp.take(A, perm, axis=0).at[idx, perm].set(1.0)
        S = jnp.zeros((pnow, A.shape[0]), jnp.bfloat16).at[:know].set(
            Sr.astype(jnp.bfloat16))
        Bc = jnp.take(A, perm, axis=1).at[perm, idx].set(1.0)
        B = jnp.zeros((A.shape[0], pnow), jnp.bfloat16).at[:, :know].set(
            Bc.astype(jnp.bfloat16))
        return _mm(S, B, zero_diag=True)

    vals2, perm2 = pool_sel(h1, k1, pool_p[1], k2)
    A2 = next_A(A1, perm2, k2, p2)
    deg2 = _mm(jnp.ones((8, p2), jnp.float32), A2)[0] + 2.0
    dis2 = jax.lax.rsqrt(deg2)
    x2 = jnp.zeros((p2, h1.shape[1]), jnp.float32).at[:k2].set(
        jnp.take(h1, perm2, axis=0) * vals2[:, None])
    h2 = _gcn_agg(A2, _xw_scale(x2, down_W[2], dis2), dis2, down_b[2],
                  relu=True)

    vals3, perm3 = pool_sel(h2, k2, pool_p[2], k3)
    A3 = next_A(A2, perm3, k3, p3)
    deg3 = _mm(jnp.ones((8, p3), jnp.float32), A3)[0] + 2.0
    dis3 = jax.lax.rsqrt(deg3)
    x3 = jnp.zeros((p3, h2.shape[1]), jnp.float32).at[:k3].set(
        jnp.take(h2, perm3, axis=0) * vals3[:, None])
    h3 = _gcn_agg(A3, _xw_scale(x3, down_W[3], dis3), dis3, down_b[3],
                  relu=True)

    xu = h2 + jnp.zeros_like(h2).at[perm3].set(h3[:k3])
    hu2 = _gcn_agg(A2, _xw_scale(xu, up_W[0], dis2), dis2, up_b[0],
                   relu=True)
    xu = h1 + jnp.zeros_like(h1).at[perm2].set(hu2[:k2])
    hu1 = _gcn_agg(A1, _xw_scale(xu, up_W[1], dis1), dis1, up_b[1],
                   relu=True)
    xu = h0 + jnp.zeros_like(h0).at[perm1].set(hu1[:k1])
    return gcn0(xu, up_W[2], up_b[2], relu=False)


# speedup vs baseline: 1.3969x; 1.2150x over previous
import functools

import numpy as np
import jax
import jax.numpy as jnp
from jax.experimental import pallas as pl
from jax.experimental.pallas import tpu as pltpu


def _round_up(v, m):
    return (v + m - 1) // m * m


def _pick(dim, cands):
    for c in cands:
        if dim % c == 0:
            return c
    raise ValueError(f"no block size for {dim} in {cands}")


_LANE = (2048, 1024, 512, 256, 128)
_MMROW = (1024, 512, 256, 128, 8)
_ROW = (512, 400, 256, 200, 128, 80, 40, 8)


def _mm_kernel(a_ref, b_ref, o_ref, acc_ref, *, nk, zero_diag, bm, bn):
    k = pl.program_id(2)

    @pl.when(k == 0)
    def _():
        acc_ref[...] = jnp.zeros_like(acc_ref)

    acc_ref[...] += jnp.dot(a_ref[...], b_ref[...],
                            preferred_element_type=jnp.float32)

    @pl.when(k == nk - 1)
    def _():
        out = acc_ref[...]
        if zero_diag:
            i, j = pl.program_id(0), pl.program_id(1)
            rows = jax.lax.broadcasted_iota(jnp.int32, (bm, bn), 0) + i * bm
            cols = jax.lax.broadcasted_iota(jnp.int32, (bm, bn), 1) + j * bn
            out = jnp.where(rows == cols, 0.0, out)
        o_ref[...] = out


def _mm(a, b, zero_diag=False):
    m, kdim = a.shape
    _, n = b.shape
    bm = _pick(m, _MMROW)
    bn = _pick(n, _LANE)
    bk = _pick(kdim, _LANE)
    grid = (m // bm, n // bn, kdim // bk)
    return pl.pallas_call(
        functools.partial(_mm_kernel, nk=grid[2], zero_diag=zero_diag,
                          bm=bm, bn=bn),
        grid=grid,
        in_specs=[
            pl.BlockSpec((bm, bk), lambda i, j, k: (i, k)),
            pl.BlockSpec((bk, bn), lambda i, j, k: (k, j)),
        ],
        out_specs=pl.BlockSpec((bm, bn), lambda i, j, k: (i, j)),
        out_shape=jax.ShapeDtypeStruct((m, n), jnp.float32),
        scratch_shapes=[pltpu.VMEM((bm, bn), jnp.float32)],
        compiler_params=pltpu.CompilerParams(
            dimension_semantics=("parallel", "parallel", "arbitrary")),
    )(a, b)


def _xw_kernel(x_ref, w_ref, dis_ref, o_ref):
    o_ref[...] = dis_ref[...] * jnp.dot(x_ref[...], w_ref[...],
                                        preferred_element_type=jnp.float32)


def _xw_scale(x, w, dis):
    m, d = x.shape
    h = w.shape[1]
    bm = _pick(m, _ROW)
    return pl.pallas_call(
        _xw_kernel,
        grid=(m // bm,),
        in_specs=[
            pl.BlockSpec((bm, d), lambda i: (i, 0)),
            pl.BlockSpec((d, h), lambda i: (0, 0)),
            pl.BlockSpec((bm, 1), lambda i: (i, 0)),
        ],
        out_specs=pl.BlockSpec((bm, h), lambda i: (i, 0)),
        out_shape=jax.ShapeDtypeStruct((m, h), jnp.float32),
        compiler_params=pltpu.CompilerParams(
            dimension_semantics=("parallel",)),
    )(x, w, dis[:, None])


def _agg_kernel(a_ref, tk_ref, ti_ref, dis_ref, b_ref, o_ref, acc_ref,
                *, nk, relu):
    k = pl.program_id(1)

    @pl.when(k == 0)
    def _():
        acc_ref[...] = jnp.zeros_like(acc_ref)

    acc_ref[...] += jax.lax.dot_general(
        a_ref[...], tk_ref[...], (((0,), (0,)), ((), ())),
        preferred_element_type=jnp.float32)

    @pl.when(k == nk - 1)
    def _():
        out = dis_ref[...] * (acc_ref[...] + 2.0 * ti_ref[...]) + b_ref[...]
        o_ref[...] = jnp.maximum(out, 0.0) if relu else out


def _gcn_agg(A, t, dis, b, relu):
    m = A.shape[0]
    h = t.shape[1]
    bm = _pick(m, _LANE)
    bk = _pick(m, _LANE)
    grid = (m // bm, m // bk)
    return pl.pallas_call(
        functools.partial(_agg_kernel, nk=grid[1], relu=relu),
        grid=grid,
        in_specs=[
            pl.BlockSpec((bk, bm), lambda i, k: (k, i)),
            pl.BlockSpec((bk, h), lambda i, k: (k, 0)),
            pl.BlockSpec((bm, h), lambda i, k: (i, 0)),
            pl.BlockSpec((bm, 1), lambda i, k: (i, 0)),
            pl.BlockSpec((1, h), lambda i, k: (0, 0)),
        ],
        out_specs=pl.BlockSpec((bm, h), lambda i, k: (i, 0)),
        out_shape=jax.ShapeDtypeStruct((m, h), jnp.float32),
        scratch_shapes=[pltpu.VMEM((bm, h), jnp.float32)],
        compiler_params=pltpu.CompilerParams(
            dimension_semantics=("parallel", "arbitrary")),
    )(A, t, t, dis[:, None], b[None, :])


def kernel(x, edge_index, down_W, down_b, pool_p, up_W, up_b):
    n0, _ = x.shape
    e = edge_index.shape[1]
    src, dst = edge_index[0], edge_index[1]

    ones_e = jnp.ones((e,), jnp.float32)
    cnt_dst = jax.ops.segment_sum(ones_e, dst, num_segments=n0)
    self_cnt = jax.ops.segment_sum((src == dst).astype(jnp.float32), dst,
                                   num_segments=n0)
    selfw = jnp.where(self_cnt == 0.0, 2.0, 0.0)
    deg0 = cnt_dst + selfw
    dis0 = jnp.where(deg0 > 0.0, jax.lax.rsqrt(deg0), 0.0)

    def gcn0(xin, W, b, relu):
        t = _xw_scale(xin, W, dis0)
        seg = jax.ops.segment_sum(jnp.take(t, src, axis=0), dst,
                                  num_segments=n0)
        out = dis0[:, None] * (seg + selfw[:, None] * t) + b[None, :]
        return jnp.maximum(out, 0.0) if relu else out

    def pool_sel(h, nreal, p, know):
        score = jnp.tanh(jnp.dot(h[:nreal], p) / jnp.linalg.norm(p))
        vals, perm = jax.lax.top_k(score, know)
        return vals, perm

    k1 = int(np.ceil(0.5 * n0))
    k2 = int(np.ceil(0.5 * k1))
    k3 = int(np.ceil(0.5 * k2))
    p1, p2, p3 = _round_up(k1, 128), _round_up(k2, 128), _round_up(k3, 128)
    pk0 = _round_up(n0, 128)

    h0 = gcn0(x, down_W[0], down_b[0], relu=True)

    vals1, perm1 = pool_sel(h0, n0, pool_p[0], k1)
    inv1 = jnp.full((n0,), -1, jnp.int32).at[perm1].set(
        jnp.arange(k1, dtype=jnp.int32))
    nonself = src != dst
    iota1 = jnp.arange(k1, dtype=jnp.int32)
    okB = nonself & (inv1[dst] >= 0)
    rowB = jnp.concatenate([src, perm1])
    colB = jnp.concatenate([jnp.where(okB, inv1[dst], p1), iota1])
    B1 = jnp.zeros((pk0, p1), jnp.float32).at[rowB, colB].add(
        1.0, mode="drop").astype(jnp.bfloat16)
    okS = nonself & (inv1[src] >= 0)
    rowS = jnp.concatenate([jnp.where(okS, inv1[src], p1), iota1])
    colS = jnp.concatenate([dst, perm1])
    S1 = jnp.zeros((p1, pk0), jnp.float32).at[rowS, colS].add(
        1.0, mode="drop").astype(jnp.bfloat16)
    A1 = _mm(S1, B1, zero_diag=True)

    deg1 = _mm(jnp.ones((8, p1), jnp.float32), A1)[0] + 2.0
    dis1 = jax.lax.rsqrt(deg1)
    x1 = jnp.zeros((p1, h0.shape[1]), jnp.float32).at[:k1].set(
        jnp.take(h0, perm1, axis=0) * vals1[:, None])
    h1 = _gcn_agg(A1, _xw_scale(x1, down_W[1], dis1), dis1, down_b[1],
                  relu=True)

    def next_A(A, perm, know, pnow):
        idx = jnp.arange(know)
        Sr = jnp.take(A, perm, axis=0).at[idx, perm].set(1.0)
        S = jnp.zeros((pnow, A.shape[0]), jnp.bfloat16).at[:know].set(
            Sr.astype(jnp.bfloat16))
        Bc = jnp.take(A, perm, axis=1).at[perm, idx].set(1.0)
        B = jnp.zeros((A.shape[0], pnow), jnp.bfloat16).at[:, :know].set(
            Bc.astype(jnp.bfloat16))
        return _mm(S, B, zero_diag=True)

    vals2, perm2 = pool_sel(h1, k1, pool_p[1], k2)
    A2 = next_A(A1, perm2, k2, p2)
    deg2 = _mm(jnp.ones((8, p2), jnp.float32), A2)[0] + 2.0
    dis2 = jax.lax.rsqrt(deg2)
    x2 = jnp.zeros((p2, h1.shape[1]), jnp.float32).at[:k2].set(
        jnp.take(h1, perm2, axis=0) * vals2[:, None])
    h2 = _gcn_agg(A2, _xw_scale(x2, down_W[2], dis2), dis2, down_b[2],
                  relu=True)

    vals3, perm3 = pool_sel(h2, k2, pool_p[2], k3)
    A3 = next_A(A2, perm3, k3, p3)
    deg3 = _mm(jnp.ones((8, p3), jnp.float32), A3)[0] + 2.0
    dis3 = jax.lax.rsqrt(deg3)
    x3 = jnp.zeros((p3, h2.shape[1]), jnp.float32).at[:k3].set(
        jnp.take(h2, perm3, axis=0) * vals3[:, None])
    h3 = _gcn_agg(A3, _xw_scale(x3, down_W[3], dis3), dis3, down_b[3],
                  relu=True)

    xu = h2 + jnp.zeros_like(h2).at[perm3].set(h3[:k3])
    hu2 = _gcn_agg(A2, _xw_scale(xu, up_W[0], dis2), dis2, up_b[0],
                   relu=True)
    xu = h1 + jnp.zeros_like(h1).at[perm2].set(hu2[:k2])
    hu1 = _gcn_agg(A1, _xw_scale(xu, up_W[1], dis1), dis1, up_b[1],
                   relu=True)
    xu = h0 + jnp.zeros_like(h0).at[perm1].set(hu1[:k1])
    return gcn0(xu, up_W[2], up_b[2], relu=False)


# speedup vs baseline: 1.6472x; 1.1792x over previous
import functools

import numpy as np
import jax
import jax.numpy as jnp
from jax.experimental import pallas as pl
from jax.experimental.pallas import tpu as pltpu
from jax.experimental.pallas import tpu_sc as plsc


def _round_up(v, m):
    return (v + m - 1) // m * m


def _pick(dim, cands):
    for c in cands:
        if dim % c == 0:
            return c
    raise ValueError(f"no block size for {dim} in {cands}")


_LANE = (2048, 1024, 512, 256, 128)
_MMROW = (1024, 512, 256, 128, 8)
_ROW = (512, 400, 256, 200, 128, 80, 40, 8)


def _mm_kernel(a_ref, b_ref, o_ref, acc_ref, *, nk, zero_diag, bm, bn):
    k = pl.program_id(2)

    @pl.when(k == 0)
    def _():
        acc_ref[...] = jnp.zeros_like(acc_ref)

    acc_ref[...] += jnp.dot(a_ref[...], b_ref[...],
                            preferred_element_type=jnp.float32)

    @pl.when(k == nk - 1)
    def _():
        out = acc_ref[...]
        if zero_diag:
            i, j = pl.program_id(0), pl.program_id(1)
            rows = jax.lax.broadcasted_iota(jnp.int32, (bm, bn), 0) + i * bm
            cols = jax.lax.broadcasted_iota(jnp.int32, (bm, bn), 1) + j * bn
            out = jnp.where(rows == cols, 0.0, out)
        o_ref[...] = out


def _mm(a, b, zero_diag=False):
    m, kdim = a.shape
    _, n = b.shape
    bm = _pick(m, _MMROW)
    bn = _pick(n, _LANE)
    bk = _pick(kdim, _LANE)
    grid = (m // bm, n // bn, kdim // bk)
    return pl.pallas_call(
        functools.partial(_mm_kernel, nk=grid[2], zero_diag=zero_diag,
                          bm=bm, bn=bn),
        grid=grid,
        in_specs=[
            pl.BlockSpec((bm, bk), lambda i, j, k: (i, k)),
            pl.BlockSpec((bk, bn), lambda i, j, k: (k, j)),
        ],
        out_specs=pl.BlockSpec((bm, bn), lambda i, j, k: (i, j)),
        out_shape=jax.ShapeDtypeStruct((m, n), jnp.float32),
        scratch_shapes=[pltpu.VMEM((bm, bn), jnp.float32)],
        compiler_params=pltpu.CompilerParams(
            dimension_semantics=("parallel", "parallel", "arbitrary")),
    )(a, b)


def _xw_kernel(x_ref, w_ref, dis_ref, o_ref):
    o_ref[...] = dis_ref[...] * jnp.dot(x_ref[...], w_ref[...],
                                        preferred_element_type=jnp.float32)


def _xw_scale(x, w, dis):
    m, d = x.shape
    h = w.shape[1]
    bm = _pick(m, _ROW)
    return pl.pallas_call(
        _xw_kernel,
        grid=(m // bm,),
        in_specs=[
            pl.BlockSpec((bm, d), lambda i: (i, 0)),
            pl.BlockSpec((d, h), lambda i: (0, 0)),
            pl.BlockSpec((bm, 1), lambda i: (i, 0)),
        ],
        out_specs=pl.BlockSpec((bm, h), lambda i: (i, 0)),
        out_shape=jax.ShapeDtypeStruct((m, h), jnp.float32),
        compiler_params=pltpu.CompilerParams(
            dimension_semantics=("parallel",)),
    )(x, w, dis[:, None])


def _agg_kernel(a_ref, tk_ref, ti_ref, dis_ref, b_ref, o_ref, acc_ref,
                *, nk, relu):
    k = pl.program_id(1)

    @pl.when(k == 0)
    def _():
        acc_ref[...] = jnp.zeros_like(acc_ref)

    acc_ref[...] += jax.lax.dot_general(
        a_ref[...], tk_ref[...], (((0,), (0,)), ((), ())),
        preferred_element_type=jnp.float32)

    @pl.when(k == nk - 1)
    def _():
        out = dis_ref[...] * (acc_ref[...] + 2.0 * ti_ref[...]) + b_ref[...]
        o_ref[...] = jnp.maximum(out, 0.0) if relu else out


def _gcn_agg(A, t, dis, b, relu):
    m = A.shape[0]
    h = t.shape[1]
    bm = _pick(m, _LANE)
    bk = _pick(m, _LANE)
    grid = (m // bm, m // bk)
    return pl.pallas_call(
        functools.partial(_agg_kernel, nk=grid[1], relu=relu),
        grid=grid,
        in_specs=[
            pl.BlockSpec((bk, bm), lambda i, k: (k, i)),
            pl.BlockSpec((bk, h), lambda i, k: (k, 0)),
            pl.BlockSpec((bm, h), lambda i, k: (i, 0)),
            pl.BlockSpec((bm, 1), lambda i, k: (i, 0)),
            pl.BlockSpec((1, h), lambda i, k: (0, 0)),
        ],
        out_specs=pl.BlockSpec((bm, h), lambda i, k: (i, 0)),
        out_shape=jax.ShapeDtypeStruct((m, h), jnp.float32),
        scratch_shapes=[pltpu.VMEM((bm, h), jnp.float32)],
        compiler_params=pltpu.CompilerParams(
            dimension_semantics=("parallel", "arbitrary")),
    )(A, t, t, dis[:, None], b[None, :])


_SC_CORES = 2
_SC_TILES = 16
_SEG_BATCH = 200


def _seg_body(t_hbm, src_hbm, dst_hbm, zero_hbm, out_hbm,
              idx_s, idx_d, rows, acc, sem, *, n_rows, n_edges):
    c = jax.lax.axis_index("c")
    s = jax.lax.axis_index("s")
    chunk = n_rows // _SC_TILES
    pltpu.sync_copy(zero_hbm.at[pl.ds(s * chunk, chunk)],
                    acc.at[pl.ds(s * chunk, chunk)])
    plsc.subcore_barrier()
    per_core = n_edges // _SC_CORES
    per_tile = per_core // _SC_TILES
    base = c * per_core + s * per_tile

    def body(j, carry):
        off = base + j * _SEG_BATCH
        pltpu.sync_copy(src_hbm.at[pl.ds(off, _SEG_BATCH)], idx_s)
        pltpu.sync_copy(dst_hbm.at[pl.ds(off, _SEG_BATCH)], idx_d)
        pltpu.async_copy(t_hbm.at[idx_s], rows, sem).wait()
        pltpu.sync_copy(rows, acc.at[idx_d], add=True)
        return carry

    jax.lax.fori_loop(0, per_tile // _SEG_BATCH, body, 0)
    plsc.subcore_barrier()
    pltpu.sync_copy(acc.at[pl.ds(s * chunk, chunk)],
                    out_hbm.at[c, pl.ds(s * chunk, chunk)])


def _sc_segsum(t, src_arr, dst_arr, zero_rows):
    n_rows, h = t.shape
    n_acc = zero_rows.shape[0]
    n_edges = src_arr.shape[0]
    kern = pl.kernel(
        functools.partial(_seg_body, n_rows=n_acc, n_edges=n_edges),
        mesh=plsc.VectorSubcoreMesh(core_axis_name="c", subcore_axis_name="s"),
        out_type=jax.ShapeDtypeStruct((_SC_CORES, n_acc, h), jnp.float32),
        scratch_types=[
            pltpu.VMEM((_SEG_BATCH,), jnp.int32),
            pltpu.VMEM((_SEG_BATCH,), jnp.int32),
            pltpu.VMEM((_SEG_BATCH, h), jnp.float32),
            pltpu.VMEM_SHARED((n_acc, h), jnp.float32),
            pltpu.SemaphoreType.DMA,
        ],
    )
    part = kern(t, src_arr, dst_arr, zero_rows)
    return (part[0] + part[1])[:n_rows]


def kernel(x, edge_index, down_W, down_b, pool_p, up_W, up_b):
    n0, _ = x.shape
    e = edge_index.shape[1]
    src, dst = edge_index[0], edge_index[1]

    ones_e = jnp.ones((e,), jnp.float32)
    cnt_dst = jax.ops.segment_sum(ones_e, dst, num_segments=n0)
    self_cnt = jax.ops.segment_sum((src == dst).astype(jnp.float32), dst,
                                   num_segments=n0)
    selfw = jnp.where(self_cnt == 0.0, 2.0, 0.0)
    deg0 = cnt_dst + selfw
    dis0 = jnp.where(deg0 > 0.0, jax.lax.rsqrt(deg0), 0.0)

    src_c = jnp.asarray(src)
    dst_c = jnp.asarray(dst)
    zero_rows = jnp.zeros((_round_up(n0, _SC_TILES * 8), x.shape[1]),
                          jnp.float32)

    def gcn0(xin, W, b, relu):
        t = _xw_scale(xin, W, dis0)
        seg = _sc_segsum(t, src_c, dst_c, zero_rows)
        out = dis0[:, None] * (seg + selfw[:, None] * t) + b[None, :]
        return jnp.maximum(out, 0.0) if relu else out

    def pool_sel(h, nreal, p, know):
        score = jnp.tanh(jnp.dot(h[:nreal], p) / jnp.linalg.norm(p))
        vals, perm = jax.lax.top_k(score, know)
        return vals, perm

    k1 = int(np.ceil(0.5 * n0))
    k2 = int(np.ceil(0.5 * k1))
    k3 = int(np.ceil(0.5 * k2))
    p1, p2, p3 = _round_up(k1, 128), _round_up(k2, 128), _round_up(k3, 128)
    pk0 = _round_up(n0, 128)

    h0 = gcn0(x, down_W[0], down_b[0], relu=True)

    vals1, perm1 = pool_sel(h0, n0, pool_p[0], k1)
    inv1 = jnp.full((n0,), -1, jnp.int32).at[perm1].set(
        jnp.arange(k1, dtype=jnp.int32))
    nonself = src != dst
    iota1 = jnp.arange(k1, dtype=jnp.int32)
    okB = nonself & (inv1[dst] >= 0)
    rowB = jnp.concatenate([src, perm1])
    colB = jnp.concatenate([jnp.where(okB, inv1[dst], p1), iota1])
    B1 = jnp.zeros((pk0, p1), jnp.float32).at[rowB, colB].add(
        1.0, mode="drop").astype(jnp.bfloat16)
    okS = nonself & (inv1[src] >= 0)
    rowS = jnp.concatenate([jnp.where(okS, inv1[src], p1), iota1])
    colS = jnp.concatenate([dst, perm1])
    S1 = jnp.zeros((p1, pk0), jnp.float32).at[rowS, colS].add(
        1.0, mode="drop").astype(jnp.bfloat16)
    A1 = _mm(S1, B1, zero_diag=True)

    deg1 = _mm(jnp.ones((8, p1), jnp.float32), A1)[0] + 2.0
    dis1 = jax.lax.rsqrt(deg1)
    x1 = jnp.zeros((p1, h0.shape[1]), jnp.float32).at[:k1].set(
        jnp.take(h0, perm1, axis=0) * vals1[:, None])
    h1 = _gcn_agg(A1, _xw_scale(x1, down_W[1], dis1), dis1, down_b[1],
                  relu=True)

    def next_A(A, perm, know, pnow):
        idx = jnp.arange(know)
        Sr = jnp.take(A, perm, axis=0).at[idx, perm].set(1.0)
        S = jnp.zeros((pnow, A.shape[0]), jnp.bfloat16).at[:know].set(
            Sr.astype(jnp.bfloat16))
        Bc = jnp.take(A, perm, axis=1).at[perm, idx].set(1.0)
        B = jnp.zeros((A.shape[0], pnow), jnp.bfloat16).at[:, :know].set(
            Bc.astype(jnp.bfloat16))
        return _mm(S, B, zero_diag=True)

    vals2, perm2 = pool_sel(h1, k1, pool_p[1], k2)
    A2 = next_A(A1, perm2, k2, p2)
    deg2 = _mm(jnp.ones((8, p2), jnp.float32), A2)[0] + 2.0
    dis2 = jax.lax.rsqrt(deg2)
    x2 = jnp.zeros((p2, h1.shape[1]), jnp.float32).at[:k2].set(
        jnp.take(h1, perm2, axis=0) * vals2[:, None])
    h2 = _gcn_agg(A2, _xw_scale(x2, down_W[2], dis2), dis2, down_b[2],
                  relu=True)

    vals3, perm3 = pool_sel(h2, k2, pool_p[2], k3)
    A3 = next_A(A2, perm3, k3, p3)
    deg3 = _mm(jnp.ones((8, p3), jnp.float32), A3)[0] + 2.0
    dis3 = jax.lax.rsqrt(deg3)
    x3 = jnp.zeros((p3, h2.shape[1]), jnp.float32).at[:k3].set(
        jnp.take(h2, perm3, axis=0) * vals3[:, None])
    h3 = _gcn_agg(A3, _xw_scale(x3, down_W[3], dis3), dis3, down_b[3],
                  relu=True)

    xu = h2 + jnp.zeros_like(h2).at[perm3].set(h3[:k3])
    hu2 = _gcn_agg(A2, _xw_scale(xu, up_W[0], dis2), dis2, up_b[0],
                   relu=True)
    xu = h1 + jnp.zeros_like(h1).at[perm2].set(hu2[:k2])
    hu1 = _gcn_agg(A1, _xw_scale(xu, up_W[1], dis1), dis1, up_b[1],
                   relu=True)
    xu = h0 + jnp.zeros_like(h0).at[perm1].set(hu1[:k1])
    return gcn0(xu, up_W[2], up_b[2], relu=False)


# speedup vs baseline: 1.7817x; 1.0816x over previous
import functools

import numpy as np
import jax
import jax.numpy as jnp
from jax.experimental import pallas as pl
from jax.experimental.pallas import tpu as pltpu
from jax.experimental.pallas import tpu_sc as plsc


def _round_up(v, m):
    return (v + m - 1) // m * m


def _pick(dim, cands):
    for c in cands:
        if dim % c == 0:
            return c
    raise ValueError(f"no block size for {dim} in {cands}")


_LANE = (2048, 1280, 1024, 512, 256, 128)
_MMROW = (1280, 1024, 512, 256, 128, 8)
_ROW = (512, 400, 256, 200, 128, 80, 40, 8)


def _mm_kernel(a_ref, b_ref, o_ref, acc_ref, *, nk, zero_diag, bm, bn):
    k = pl.program_id(2)

    @pl.when(k == 0)
    def _():
        acc_ref[...] = jnp.zeros_like(acc_ref)

    acc_ref[...] += jnp.dot(a_ref[...], b_ref[...],
                            preferred_element_type=jnp.float32)

    @pl.when(k == nk - 1)
    def _():
        out = acc_ref[...]
        if zero_diag:
            i, j = pl.program_id(0), pl.program_id(1)
            rows = jax.lax.broadcasted_iota(jnp.int32, (bm, bn), 0) + i * bm
            cols = jax.lax.broadcasted_iota(jnp.int32, (bm, bn), 1) + j * bn
            out = jnp.where(rows == cols, 0.0, out)
        o_ref[...] = out


def _mm(a, b, zero_diag=False):
    m, kdim = a.shape
    _, n = b.shape
    bm = _pick(m, _MMROW)
    bn = _pick(n, _LANE)
    bk = _pick(kdim, _LANE)
    grid = (m // bm, n // bn, kdim // bk)
    return pl.pallas_call(
        functools.partial(_mm_kernel, nk=grid[2], zero_diag=zero_diag,
                          bm=bm, bn=bn),
        grid=grid,
        in_specs=[
            pl.BlockSpec((bm, bk), lambda i, j, k: (i, k)),
            pl.BlockSpec((bk, bn), lambda i, j, k: (k, j)),
        ],
        out_specs=pl.BlockSpec((bm, bn), lambda i, j, k: (i, j)),
        out_shape=jax.ShapeDtypeStruct((m, n), jnp.float32),
        scratch_shapes=[pltpu.VMEM((bm, bn), jnp.float32)],
        compiler_params=pltpu.CompilerParams(
            dimension_semantics=("parallel", "parallel", "arbitrary")),
    )(a, b)


def _xw_kernel(x_ref, w_ref, dis_ref, o_ref):
    o_ref[...] = dis_ref[...] * jnp.dot(x_ref[...], w_ref[...],
                                        preferred_element_type=jnp.float32)


def _xw_scale(x, w, dis):
    m, d = x.shape
    h = w.shape[1]
    bm = _pick(m, _ROW)
    return pl.pallas_call(
        _xw_kernel,
        grid=(m // bm,),
        in_specs=[
            pl.BlockSpec((bm, d), lambda i: (i, 0)),
            pl.BlockSpec((d, h), lambda i: (0, 0)),
            pl.BlockSpec((bm, 1), lambda i: (i, 0)),
        ],
        out_specs=pl.BlockSpec((bm, h), lambda i: (i, 0)),
        out_shape=jax.ShapeDtypeStruct((m, h), jnp.float32),
        compiler_params=pltpu.CompilerParams(
            dimension_semantics=("parallel",)),
    )(x, w, dis[:, None])


def _agg_kernel(a_ref, tk_ref, ti_ref, dis_ref, b_ref, o_ref, acc_ref,
                *, nk, relu):
    k = pl.program_id(1)

    @pl.when(k == 0)
    def _():
        acc_ref[...] = jnp.zeros_like(acc_ref)

    acc_ref[...] += jax.lax.dot_general(
        a_ref[...], tk_ref[...], (((0,), (0,)), ((), ())),
        preferred_element_type=jnp.float32)

    @pl.when(k == nk - 1)
    def _():
        out = dis_ref[...] * (acc_ref[...] + 2.0 * ti_ref[...]) + b_ref[...]
        o_ref[...] = jnp.maximum(out, 0.0) if relu else out


def _gcn_agg(A, t, dis, b, relu):
    m = A.shape[0]
    h = t.shape[1]
    bm = _pick(m, _LANE)
    bk = _pick(m, _LANE)
    grid = (m // bm, m // bk)
    return pl.pallas_call(
        functools.partial(_agg_kernel, nk=grid[1], relu=relu),
        grid=grid,
        in_specs=[
            pl.BlockSpec((bk, bm), lambda i, k: (k, i)),
            pl.BlockSpec((bk, h), lambda i, k: (k, 0)),
            pl.BlockSpec((bm, h), lambda i, k: (i, 0)),
            pl.BlockSpec((bm, 1), lambda i, k: (i, 0)),
            pl.BlockSpec((1, h), lambda i, k: (0, 0)),
        ],
        out_specs=pl.BlockSpec((bm, h), lambda i, k: (i, 0)),
        out_shape=jax.ShapeDtypeStruct((m, h), jnp.float32),
        scratch_shapes=[pltpu.VMEM((bm, h), jnp.float32)],
        compiler_params=pltpu.CompilerParams(
            dimension_semantics=("parallel", "arbitrary")),
    )(A, t, t, dis[:, None], b[None, :])


_SC_CORES = 2
_SC_TILES = 16
_SEG_BATCH = 200


def _seg_body(t_hbm, src_hbm, dst_hbm, zero_hbm, out_hbm,
              idx_s, idx_d, rows, acc, sem, *, n_rows, n_edges):
    c = jax.lax.axis_index("c")
    s = jax.lax.axis_index("s")
    chunk = n_rows // _SC_TILES
    pltpu.sync_copy(zero_hbm.at[pl.ds(s * chunk, chunk)],
                    acc.at[pl.ds(s * chunk, chunk)])
    plsc.subcore_barrier()
    per_core = n_edges // _SC_CORES
    per_tile = per_core // _SC_TILES
    base = c * per_core + s * per_tile

    def body(j, carry):
        off = base + j * _SEG_BATCH
        pltpu.sync_copy(src_hbm.at[pl.ds(off, _SEG_BATCH)], idx_s)
        pltpu.sync_copy(dst_hbm.at[pl.ds(off, _SEG_BATCH)], idx_d)
        pltpu.async_copy(t_hbm.at[idx_s], rows, sem).wait()
        pltpu.sync_copy(rows, acc.at[idx_d], add=True)
        return carry

    jax.lax.fori_loop(0, per_tile // _SEG_BATCH, body, 0)
    plsc.subcore_barrier()
    pltpu.sync_copy(acc.at[pl.ds(s * chunk, chunk)],
                    out_hbm.at[c, pl.ds(s * chunk, chunk)])


def _sc_segsum(t, src_arr, dst_arr, zero_rows):
    n_rows, h = t.shape
    n_acc = zero_rows.shape[0]
    n_edges = src_arr.shape[0]
    kern = pl.kernel(
        functools.partial(_seg_body, n_rows=n_acc, n_edges=n_edges),
        mesh=plsc.VectorSubcoreMesh(core_axis_name="c", subcore_axis_name="s"),
        out_type=jax.ShapeDtypeStruct((_SC_CORES, n_acc, h), jnp.float32),
        scratch_types=[
            pltpu.VMEM((_SEG_BATCH,), jnp.int32),
            pltpu.VMEM((_SEG_BATCH,), jnp.int32),
            pltpu.VMEM((_SEG_BATCH, h), jnp.float32),
            pltpu.VMEM_SHARED((n_acc, h), jnp.float32),
            pltpu.SemaphoreType.DMA,
        ],
    )
    part = kern(t, src_arr, dst_arr, zero_rows)
    return (part[0] + part[1])[:n_rows]


def kernel(x, edge_index, down_W, down_b, pool_p, up_W, up_b):
    n0, _ = x.shape
    e = edge_index.shape[1]
    src, dst = edge_index[0], edge_index[1]

    ones_e = jnp.ones((e,), jnp.float32)
    cnt_dst = jax.ops.segment_sum(ones_e, dst, num_segments=n0)
    self_cnt = jax.ops.segment_sum((src == dst).astype(jnp.float32), dst,
                                   num_segments=n0)
    selfw = jnp.where(self_cnt == 0.0, 2.0, 0.0)
    deg0 = cnt_dst + selfw
    dis0 = jnp.where(deg0 > 0.0, jax.lax.rsqrt(deg0), 0.0)

    src_c = jnp.asarray(src)
    dst_c = jnp.asarray(dst)
    zero_rows = jnp.zeros((_round_up(n0, _SC_TILES * 8), x.shape[1]),
                          jnp.float32)

    def gcn0(xin, W, b, relu):
        t = _xw_scale(xin, W, dis0)
        seg = _sc_segsum(t, src_c, dst_c, zero_rows)
        out = dis0[:, None] * (seg + selfw[:, None] * t) + b[None, :]
        return jnp.maximum(out, 0.0) if relu else out

    def pool_sel(h, nreal, p, know):
        score = jnp.tanh(jnp.dot(h[:nreal], p) / jnp.linalg.norm(p))
        vals, perm = jax.lax.top_k(score, know)
        return vals, perm

    k1 = int(np.ceil(0.5 * n0))
    k2 = int(np.ceil(0.5 * k1))
    k3 = int(np.ceil(0.5 * k2))
    p1, p2, p3 = _round_up(k1, 128), _round_up(k2, 128), _round_up(k3, 128)
    pk0 = _round_up(n0, 128)

    h0 = gcn0(x, down_W[0], down_b[0], relu=True)

    vals1, perm1 = pool_sel(h0, n0, pool_p[0], k1)
    inv1 = jnp.full((n0,), -1, jnp.int32).at[perm1].set(
        jnp.arange(k1, dtype=jnp.int32))
    nonself = src != dst
    iota1 = jnp.arange(k1, dtype=jnp.int32)
    okB = nonself & (inv1[dst] >= 0)
    rowB = jnp.concatenate([src, perm1])
    colB = jnp.concatenate([jnp.where(okB, inv1[dst], p1), iota1])
    B1 = jnp.zeros((pk0, p1), jnp.float32).at[rowB, colB].add(
        1.0, mode="drop").astype(jnp.bfloat16)
    okS = nonself & (inv1[src] >= 0)
    rowS = jnp.concatenate([jnp.where(okS, inv1[src], p1), iota1])
    colS = jnp.concatenate([dst, perm1])
    S1 = jnp.zeros((p1, pk0), jnp.float32).at[rowS, colS].add(
        1.0, mode="drop").astype(jnp.bfloat16)
    A1 = _mm(S1, B1, zero_diag=True)

    deg1 = _mm(jnp.ones((8, p1), jnp.float32), A1)[0] + 2.0
    dis1 = jax.lax.rsqrt(deg1)
    x1 = jnp.zeros((p1, h0.shape[1]), jnp.float32).at[:k1].set(
        jnp.take(h0, perm1, axis=0) * vals1[:, None])
    h1 = _gcn_agg(A1, _xw_scale(x1, down_W[1], dis1), dis1, down_b[1],
                  relu=True)

    def next_A(A, perm, know, pnow):
        idx = jnp.arange(know)
        Sr = jnp.take(A, perm, axis=0).at[idx, perm].set(1.0)
        S = jnp.zeros((pnow, A.shape[0]), jnp.bfloat16).at[:know].set(
            Sr.astype(jnp.bfloat16))
        Bc = jnp.take(A, perm, axis=1).at[perm, idx].set(1.0)
        B = jnp.zeros((A.shape[0], pnow), jnp.bfloat16).at[:, :know].set(
            Bc.astype(jnp.bfloat16))
        return _mm(S, B, zero_diag=True)

    vals2, perm2 = pool_sel(h1, k1, pool_p[1], k2)
    A2 = next_A(A1, perm2, k2, p2)
    deg2 = _mm(jnp.ones((8, p2), jnp.float32), A2)[0] + 2.0
    dis2 = jax.lax.rsqrt(deg2)
    x2 = jnp.zeros((p2, h1.shape[1]), jnp.float32).at[:k2].set(
        jnp.take(h1, perm2, axis=0) * vals2[:, None])
    h2 = _gcn_agg(A2, _xw_scale(x2, down_W[2], dis2), dis2, down_b[2],
                  relu=True)

    vals3, perm3 = pool_sel(h2, k2, pool_p[2], k3)
    A3 = next_A(A2, perm3, k3, p3)
    deg3 = _mm(jnp.ones((8, p3), jnp.float32), A3)[0] + 2.0
    dis3 = jax.lax.rsqrt(deg3)
    x3 = jnp.zeros((p3, h2.shape[1]), jnp.float32).at[:k3].set(
        jnp.take(h2, perm3, axis=0) * vals3[:, None])
    h3 = _gcn_agg(A3, _xw_scale(x3, down_W[3], dis3), dis3, down_b[3],
                  relu=True)

    xu = h2 + jnp.zeros_like(h2).at[perm3].set(h3[:k3])
    hu2 = _gcn_agg(A2, _xw_scale(xu, up_W[0], dis2), dis2, up_b[0],
                   relu=True)
    xu = h1 + jnp.zeros_like(h1).at[perm2].set(hu2[:k2])
    hu1 = _gcn_agg(A1, _xw_scale(xu, up_W[1], dis1), dis1, up_b[1],
                   relu=True)
    xu = h0 + jnp.zeros_like(h0).at[perm1].set(hu1[:k1])
    return gcn0(xu, up_W[2], up_b[2], relu=False)
